# initial kernel scaffold (unmeasured)
import jax
import jax.numpy as jnp
from jax import lax
from jax.experimental import pallas as pl
from jax.experimental.pallas import tpu as pltpu

N_DEV = 4
SQ = 1024
DM = 1024
HQ = 8
DH = 128
BLK = 256
WIN = 512
SCALE = 0.08838834764831843
WIN0 = (0, 128, 384, 512)


def _body(x_ref, wq_ref, kt_ref, vt_ref, wo_ref, out_ref,
          xg, pacc, rbuf, sbuf, ag_send, ag_recv, rs_send, rs_recv):
    i = lax.axis_index("i")
    right = lax.rem(i + 1, N_DEV)
    left = lax.rem(i + 3, N_DEV)

    barrier = pltpu.get_barrier_semaphore()
    for nbr in (left, right):
        pl.semaphore_signal(barrier, inc=1, device_id=(nbr,),
                            device_id_type=pl.DeviceIdType.MESH)
    pl.semaphore_wait(barrier, 2)

    xg[0] = x_ref[...]

    for h in range(N_DEV - 1):
        rdma = pltpu.make_async_remote_copy(
            src_ref=xg.at[h], dst_ref=xg.at[h + 1],
            send_sem=ag_send.at[h], recv_sem=ag_recv.at[h],
            device_id=(right,), device_id_type=pl.DeviceIdType.MESH)
        rdma.start()
        rdma.wait()

    def partial(s):
        xs = xg[s]
        q = jnp.dot(xs, wq_ref[...],
                    preferred_element_type=jnp.float32).astype(jnp.bfloat16)
        for r in range(SQ // BLK):
            w0 = WIN0[r]
            qi = BLK * r + lax.broadcasted_iota(jnp.int32, (BLK, WIN), 0)
            kj = w0 + lax.broadcasted_iota(jnp.int32, (BLK, WIN), 1)
            mask = jnp.abs(qi - kj) <= 128
            acc = jnp.zeros((BLK, DM), jnp.float32)
            for h in range(HQ):
                qb = q[BLK * r:BLK * (r + 1), DH * h:DH * (h + 1)]
                kb = kt_ref[s, h, w0:w0 + WIN, :]
                vb = vt_ref[s, h, w0:w0 + WIN, :]
                sc = lax.dot_general(
                    qb, kb, (((1,), (1,)), ((), ())),
                    preferred_element_type=jnp.float32) * SCALE
                sc = jnp.where(mask, sc, -1e9)
                m = jnp.max(sc, axis=-1, keepdims=True)
                e = jnp.exp(sc - m)
                w = (e / jnp.sum(e, axis=-1, keepdims=True)).astype(jnp.bfloat16)
                ctx = jnp.dot(w, vb,
                              preferred_element_type=jnp.float32
                              ).astype(jnp.bfloat16)
                acc = acc + jnp.dot(ctx, wo_ref[DH * h:DH * (h + 1), :],
                                    preferred_element_type=jnp.float32)
            pacc[s, BLK * r:BLK * (r + 1), :] = acc.astype(jnp.bfloat16)

    for s in range(N_DEV):
        partial(s)

    def rs_step(k, src):
        rdma = pltpu.make_async_remote_copy(
            src_ref=src, dst_ref=rbuf.at[k],
            send_sem=rs_send.at[k], recv_sem=rs_recv.at[k],
            device_id=(right,), device_id_type=pl.DeviceIdType.MESH)
        rdma.start()
        rdma.wait()

    rs_step(0, pacc.at[1])
    sbuf[0] = rbuf[0] + pacc[2]
    rs_step(1, sbuf.at[0])
    sbuf[1] = rbuf[1] + pacc[3]
    rs_step(2, sbuf.at[1])
    out_ref[...] = rbuf[2].astype(jnp.float32) + pacc[0].astype(jnp.float32)


def kernel(x, Wq, K_ext, V_ext, Wo):
    i = lax.axis_index("i")
    x0 = x[0].astype(jnp.bfloat16)
    wq = Wq.astype(jnp.bfloat16)
    wo = Wo.astype(jnp.bfloat16)
    idx = jnp.mod(i - jnp.arange(N_DEV), N_DEV)
    kt = jnp.transpose(
        lax.dynamic_slice_in_dim(K_ext, i * HQ, HQ, axis=2),
        (0, 2, 1, 3)).astype(jnp.bfloat16)[idx]
    vt = jnp.transpose(
        lax.dynamic_slice_in_dim(V_ext, i * HQ, HQ, axis=2),
        (0, 2, 1, 3)).astype(jnp.bfloat16)[idx]

    out2d = pl.pallas_call(
        _body,
        out_shape=jax.ShapeDtypeStruct((SQ, DM), jnp.float32),
        in_specs=[pl.BlockSpec(memory_space=pltpu.VMEM)] * 5,
        out_specs=pl.BlockSpec(memory_space=pltpu.VMEM),
        scratch_shapes=[
            pltpu.VMEM((N_DEV, SQ, DM), jnp.bfloat16),
            pltpu.VMEM((N_DEV, SQ, DM), jnp.bfloat16),
            pltpu.VMEM((N_DEV - 1, SQ, DM), jnp.bfloat16),
            pltpu.VMEM((2, SQ, DM), jnp.bfloat16),
            pltpu.SemaphoreType.DMA((N_DEV - 1,)),
            pltpu.SemaphoreType.DMA((N_DEV - 1,)),
            pltpu.SemaphoreType.DMA((N_DEV - 1,)),
            pltpu.SemaphoreType.DMA((N_DEV - 1,)),
        ],
        compiler_params=pltpu.CompilerParams(collective_id=0),
    )(x0, wq, kt, vt, wo)
    return out2d[None].astype(jnp.float32)


# baseline (device time: 300941 ns/iter reference)
import jax
import jax.numpy as jnp
from jax import lax
from jax.experimental import pallas as pl
from jax.experimental.pallas import tpu as pltpu

N_DEV = 4
SQ = 1024
DM = 1024
HQ = 8
DH = 128
BLK = 256
WIN = 512
SCALE = 0.08838834764831843
WIN0 = (0, 128, 384, 512)


def _body(x_ref, wq_ref, kt_ref, vt_ref, wo_ref, out_ref,
          xg, pacc, ag_send, ag_recv, rs_send, rs_recv, rs_ready):
    i = lax.axis_index("i")
    right = lax.rem(i + 1, N_DEV)
    left = lax.rem(i + 3, N_DEV)

    barrier = pltpu.get_barrier_semaphore()
    for nbr in (left, right):
        pl.semaphore_signal(barrier, inc=1, device_id=(nbr,),
                            device_id_type=pl.DeviceIdType.MESH)
    pl.semaphore_wait(barrier, 2)

    xg[0] = x_ref[...]

    for h in range(N_DEV - 1):
        rdma = pltpu.make_async_remote_copy(
            src_ref=xg.at[h], dst_ref=xg.at[h + 1],
            send_sem=ag_send.at[h], recv_sem=ag_recv.at[h],
            device_id=(right,), device_id_type=pl.DeviceIdType.MESH)
        rdma.start()
        rdma.wait()

    def partial(s):
        xs = xg[s]
        q = jnp.dot(xs, wq_ref[...],
                    preferred_element_type=jnp.float32).astype(jnp.bfloat16)
        for r in range(SQ // BLK):
            w0 = WIN0[r]
            qi = BLK * r + lax.broadcasted_iota(jnp.int32, (BLK, WIN), 0)
            kj = w0 + lax.broadcasted_iota(jnp.int32, (BLK, WIN), 1)
            mask = jnp.abs(qi - kj) <= 128
            acc = jnp.zeros((BLK, DM), jnp.float32)
            for h in range(HQ):
                qb = q[BLK * r:BLK * (r + 1), DH * h:DH * (h + 1)]
                kb = kt_ref[s, h, w0:w0 + WIN, :]
                vb = vt_ref[s, h, w0:w0 + WIN, :]
                sc = lax.dot_general(
                    qb, kb, (((1,), (1,)), ((), ())),
                    preferred_element_type=jnp.float32) * SCALE
                sc = jnp.where(mask, sc, -1e9)
                m = jnp.max(sc, axis=-1, keepdims=True)
                e = jnp.exp(sc - m)
                w = (e / jnp.sum(e, axis=-1, keepdims=True)).astype(jnp.bfloat16)
                ctx = jnp.dot(w, vb,
                              preferred_element_type=jnp.float32
                              ).astype(jnp.bfloat16)
                acc = acc + jnp.dot(ctx, wo_ref[DH * h:DH * (h + 1), :],
                                    preferred_element_type=jnp.float32)
            pacc[s, BLK * r:BLK * (r + 1), :] = acc.astype(jnp.bfloat16)

    for s in range(N_DEV):
        partial(s)

    pl.semaphore_signal(rs_ready, inc=1, device_id=(left,),
                        device_id_type=pl.DeviceIdType.MESH)
    pl.semaphore_wait(rs_ready, 1)

    def rs_step(k):
        rdma = pltpu.make_async_remote_copy(
            src_ref=pacc.at[1], dst_ref=xg.at[k + 1],
            send_sem=rs_send.at[k], recv_sem=rs_recv.at[k],
            device_id=(right,), device_id_type=pl.DeviceIdType.MESH)
        rdma.start()
        rdma.wait()

    rs_step(0)
    pacc[1] = xg[1] + pacc[2]
    rs_step(1)
    pacc[1] = xg[2] + pacc[3]
    rs_step(2)
    out_ref[...] = xg[3].astype(jnp.float32) + pacc[0].astype(jnp.float32)


def kernel(x, Wq, K_ext, V_ext, Wo):
    i = lax.axis_index("i")
    x0 = x[0].astype(jnp.bfloat16)
    wq = Wq.astype(jnp.bfloat16)
    wo = Wo.astype(jnp.bfloat16)
    idx = jnp.mod(i - jnp.arange(N_DEV), N_DEV)
    kt = jnp.transpose(
        lax.dynamic_slice_in_dim(K_ext, i * HQ, HQ, axis=2),
        (0, 2, 1, 3)).astype(jnp.bfloat16)[idx]
    vt = jnp.transpose(
        lax.dynamic_slice_in_dim(V_ext, i * HQ, HQ, axis=2),
        (0, 2, 1, 3)).astype(jnp.bfloat16)[idx]

    out2d = pl.pallas_call(
        _body,
        out_shape=jax.ShapeDtypeStruct((SQ, DM), jnp.float32),
        in_specs=[pl.BlockSpec(memory_space=pltpu.VMEM)] * 5,
        out_specs=pl.BlockSpec(memory_space=pltpu.VMEM),
        scratch_shapes=[
            pltpu.VMEM((N_DEV, SQ, DM), jnp.bfloat16),
            pltpu.VMEM((N_DEV, SQ, DM), jnp.bfloat16),
            pltpu.SemaphoreType.DMA((N_DEV - 1,)),
            pltpu.SemaphoreType.DMA((N_DEV - 1,)),
            pltpu.SemaphoreType.DMA((N_DEV - 1,)),
            pltpu.SemaphoreType.DMA((N_DEV - 1,)),
            pltpu.SemaphoreType.REGULAR,
        ],
        compiler_params=pltpu.CompilerParams(
            collective_id=0, vmem_limit_bytes=40 * 1024 * 1024),
    )(x0, wq, kt, vt, wo)
    return out2d[None].astype(jnp.float32)


# device time: 150907 ns/iter; 1.9942x vs baseline; 1.9942x over previous
import jax
import jax.numpy as jnp
from jax import lax
from jax.experimental import pallas as pl
from jax.experimental.pallas import tpu as pltpu

N_DEV = 4
SQ = 1024
HALF = SQ // 2
DM = 1024
HQ = 8
DH = 128
BLK = 256
WIN = 512
SCALE = 0.08838834764831843
WIN0 = (0, 128, 384, 512)


def _body(x_ref, wq_ref, kt_ref, vt_ref, wo_ref, out_ref,
          xg_t, xg_b, pacc_t, pacc_b, rbuf_t, rbuf_b,
          ag_send_t, ag_recv_t, ag_send_b, ag_recv_b,
          rs_send_t, rs_recv_t, rs_send_b, rs_recv_b):
    i = lax.axis_index("i")
    right = lax.rem(i + 1, N_DEV)
    left = lax.rem(i + 3, N_DEV)

    barrier = pltpu.get_barrier_semaphore()
    for nbr in (left, right):
        pl.semaphore_signal(barrier, inc=1, device_id=(nbr,),
                            device_id_type=pl.DeviceIdType.MESH)
    pl.semaphore_wait(barrier, 2)

    xg_t[0] = x_ref[0:HALF, :]
    xg_b[0] = x_ref[HALF:, :]

    def ag(h):
        t = pltpu.make_async_remote_copy(
            src_ref=xg_t.at[h], dst_ref=xg_t.at[h + 1],
            send_sem=ag_send_t.at[h], recv_sem=ag_recv_t.at[h],
            device_id=(right,), device_id_type=pl.DeviceIdType.MESH)
        b = pltpu.make_async_remote_copy(
            src_ref=xg_b.at[h], dst_ref=xg_b.at[h + 1],
            send_sem=ag_send_b.at[h], recv_sem=ag_recv_b.at[h],
            device_id=(left,), device_id_type=pl.DeviceIdType.MESH)
        t.start()
        b.start()
        return t, b

    def rs(k):
        t = pltpu.make_async_remote_copy(
            src_ref=pacc_t.at[1], dst_ref=rbuf_t.at[k],
            send_sem=rs_send_t.at[k], recv_sem=rs_recv_t.at[k],
            device_id=(right,), device_id_type=pl.DeviceIdType.MESH)
        b = pltpu.make_async_remote_copy(
            src_ref=pacc_b.at[1], dst_ref=rbuf_b.at[k],
            send_sem=rs_send_b.at[k], recv_sem=rs_recv_b.at[k],
            device_id=(left,), device_id_type=pl.DeviceIdType.MESH)
        t.start()
        b.start()
        return t, b

    def wait(pair):
        pair[0].wait()
        pair[1].wait()

    def attn_half(s, top):
        xg = xg_t if top else xg_b
        pacc = pacc_t if top else pacc_b
        kv_s = s if top else (N_DEV - s) % N_DEV
        q = jnp.dot(xg[s], wq_ref[...],
                    preferred_element_type=jnp.float32).astype(jnp.bfloat16)
        for rl in range(HALF // BLK):
            r = rl if top else rl + 2
            w0 = WIN0[r]
            qi = BLK * r + lax.broadcasted_iota(jnp.int32, (BLK, WIN), 0)
            kj = w0 + lax.broadcasted_iota(jnp.int32, (BLK, WIN), 1)
            mask = jnp.abs(qi - kj) <= 128
            acc = jnp.zeros((BLK, DM), jnp.float32)
            for h in range(HQ):
                qb = q[BLK * rl:BLK * (rl + 1), DH * h:DH * (h + 1)]
                kb = kt_ref[kv_s, h, w0:w0 + WIN, :]
                vb = vt_ref[kv_s, h, w0:w0 + WIN, :]
                sc = lax.dot_general(
                    qb, kb, (((1,), (1,)), ((), ())),
                    preferred_element_type=jnp.float32)
                e = jnp.exp(jnp.where(mask, sc, -1e9))
                w = (e / jnp.sum(e, axis=-1, keepdims=True)).astype(jnp.bfloat16)
                ctx = jnp.dot(w, vb,
                              preferred_element_type=jnp.float32
                              ).astype(jnp.bfloat16)
                acc = acc + jnp.dot(ctx, wo_ref[DH * h:DH * (h + 1), :],
                                    preferred_element_type=jnp.float32)
            pacc[s, BLK * rl:BLK * (rl + 1), :] = acc.astype(jnp.bfloat16)

    ag0 = ag(0)
    attn_half(0, top=True)
    wait(ag0)
    ag1 = ag(1)
    attn_half(1, top=True)
    attn_half(1, top=False)
    rs0 = rs(0)
    wait(ag1)
    ag2 = ag(2)
    attn_half(2, top=True)
    attn_half(2, top=False)
    wait(rs0)
    pacc_t[1] = rbuf_t[0] + pacc_t[2]
    pacc_b[1] = rbuf_b[0] + pacc_b[2]
    rs1 = rs(1)
    wait(ag2)
    attn_half(3, top=True)
    attn_half(3, top=False)
    wait(rs1)
    pacc_t[1] = rbuf_t[1] + pacc_t[3]
    pacc_b[1] = rbuf_b[1] + pacc_b[3]
    rs2 = rs(2)
    attn_half(0, top=False)
    wait(rs2)
    out_ref[0:HALF, :] = (rbuf_t[2].astype(jnp.float32)
                          + pacc_t[0].astype(jnp.float32))
    out_ref[HALF:, :] = (rbuf_b[2].astype(jnp.float32)
                         + pacc_b[0].astype(jnp.float32))


def kernel(x, Wq, K_ext, V_ext, Wo):
    i = lax.axis_index("i")
    x0 = x[0].astype(jnp.bfloat16)
    wq = (Wq * SCALE).astype(jnp.bfloat16)
    wo = Wo.astype(jnp.bfloat16)
    idx = jnp.mod(i - jnp.arange(N_DEV), N_DEV)
    kt = jnp.transpose(
        lax.dynamic_slice_in_dim(K_ext, i * HQ, HQ, axis=2),
        (0, 2, 1, 3)).astype(jnp.bfloat16)[idx]
    vt = jnp.transpose(
        lax.dynamic_slice_in_dim(V_ext, i * HQ, HQ, axis=2),
        (0, 2, 1, 3)).astype(jnp.bfloat16)[idx]

    half = (HALF, DM)
    out2d = pl.pallas_call(
        _body,
        out_shape=jax.ShapeDtypeStruct((SQ, DM), jnp.float32),
        in_specs=[pl.BlockSpec(memory_space=pltpu.VMEM)] * 5,
        out_specs=pl.BlockSpec(memory_space=pltpu.VMEM),
        scratch_shapes=[
            pltpu.VMEM((N_DEV, *half), jnp.bfloat16),
            pltpu.VMEM((N_DEV, *half), jnp.bfloat16),
            pltpu.VMEM((N_DEV, *half), jnp.bfloat16),
            pltpu.VMEM((N_DEV, *half), jnp.bfloat16),
            pltpu.VMEM((N_DEV - 1, *half), jnp.bfloat16),
            pltpu.VMEM((N_DEV - 1, *half), jnp.bfloat16),
            pltpu.SemaphoreType.DMA((N_DEV - 1,)),
            pltpu.SemaphoreType.DMA((N_DEV - 1,)),
            pltpu.SemaphoreType.DMA((N_DEV - 1,)),
            pltpu.SemaphoreType.DMA((N_DEV - 1,)),
            pltpu.SemaphoreType.DMA((N_DEV - 1,)),
            pltpu.SemaphoreType.DMA((N_DEV - 1,)),
            pltpu.SemaphoreType.DMA((N_DEV - 1,)),
            pltpu.SemaphoreType.DMA((N_DEV - 1,)),
        ],
        compiler_params=pltpu.CompilerParams(
            collective_id=0, vmem_limit_bytes=42 * 1024 * 1024 + 512 * 1024),
    )(x0, wq, kt, vt, wo)
    return out2d[None].astype(jnp.float32)


# device time: 145269 ns/iter; 2.0716x vs baseline; 1.0388x over previous
import jax
import jax.numpy as jnp
from jax import lax
from jax.experimental import pallas as pl
from jax.experimental.pallas import tpu as pltpu

N_DEV = 4
SQ = 1024
HALF = SQ // 2
DM = 1024
HQ = 8
DH = 128
BLK = 256
WIN = 512
SCALE = 0.08838834764831843
WIN0 = (0, 128, 384, 512)


def _body(x_ref, wq_ref, kt_ref, vt_ref, wo_ref, out_ref,
          xg_t, xg_b, pacc_t, pacc_b, rbuf_t, rbuf_b,
          ag_send_t, ag_recv_t, ag_send_b, ag_recv_b,
          rs_send_t, rs_recv_t, rs_send_b, rs_recv_b):
    i = lax.axis_index("i")
    right = lax.rem(i + 1, N_DEV)
    left = lax.rem(i + 3, N_DEV)

    barrier = pltpu.get_barrier_semaphore()
    for nbr in (left, right):
        pl.semaphore_signal(barrier, inc=1, device_id=(nbr,),
                            device_id_type=pl.DeviceIdType.MESH)
    pl.semaphore_wait(barrier, 2)

    xg_t[0] = x_ref[0:HALF, :]
    xg_b[0] = x_ref[HALF:, :]

    def ag(h):
        t = pltpu.make_async_remote_copy(
            src_ref=xg_t.at[h], dst_ref=xg_t.at[h + 1],
            send_sem=ag_send_t.at[h], recv_sem=ag_recv_t.at[h],
            device_id=(right,), device_id_type=pl.DeviceIdType.MESH)
        b = pltpu.make_async_remote_copy(
            src_ref=xg_b.at[h], dst_ref=xg_b.at[h + 1],
            send_sem=ag_send_b.at[h], recv_sem=ag_recv_b.at[h],
            device_id=(left,), device_id_type=pl.DeviceIdType.MESH)
        t.start()
        b.start()
        return t, b

    def rs(k):
        t = pltpu.make_async_remote_copy(
            src_ref=pacc_t.at[1], dst_ref=rbuf_t.at[k],
            send_sem=rs_send_t.at[k], recv_sem=rs_recv_t.at[k],
            device_id=(right,), device_id_type=pl.DeviceIdType.MESH)
        b = pltpu.make_async_remote_copy(
            src_ref=pacc_b.at[1], dst_ref=rbuf_b.at[k],
            send_sem=rs_send_b.at[k], recv_sem=rs_recv_b.at[k],
            device_id=(left,), device_id_type=pl.DeviceIdType.MESH)
        t.start()
        b.start()
        return t, b

    def wait(pair):
        pair[0].wait()
        pair[1].wait()

    def attn_half(s, top):
        xg = xg_t if top else xg_b
        pacc = pacc_t if top else pacc_b
        i = lax.axis_index("i")
        b = lax.rem(i + (N_DEV - s if top else s), N_DEV)
        q = jnp.dot(xg[s], wq_ref[...],
                    preferred_element_type=jnp.float32).astype(jnp.bfloat16)
        for rl in range(HALF // BLK):
            r = rl if top else rl + 2
            w0 = WIN0[r]
            qi = BLK * r + lax.broadcasted_iota(jnp.int32, (BLK, WIN), 0)
            kj = w0 + lax.broadcasted_iota(jnp.int32, (BLK, WIN), 1)
            mask = jnp.abs(qi - kj) <= 128
            acc = jnp.zeros((BLK, DM), jnp.float32)
            for h in range(HQ):
                qb = q[BLK * rl:BLK * (rl + 1), DH * h:DH * (h + 1)]
                kb = kt_ref[b, w0:w0 + WIN, DH * h:DH * (h + 1)]
                vb = vt_ref[b, w0:w0 + WIN, DH * h:DH * (h + 1)]
                sc = lax.dot_general(
                    qb, kb, (((1,), (1,)), ((), ())),
                    preferred_element_type=jnp.float32)
                e = jnp.exp(jnp.where(mask, sc, -1e9))
                w = (e / jnp.sum(e, axis=-1, keepdims=True)).astype(jnp.bfloat16)
                ctx = jnp.dot(w, vb,
                              preferred_element_type=jnp.float32
                              ).astype(jnp.bfloat16)
                acc = acc + jnp.dot(ctx, wo_ref[DH * h:DH * (h + 1), :],
                                    preferred_element_type=jnp.float32)
            pacc[s, BLK * rl:BLK * (rl + 1), :] = acc.astype(jnp.bfloat16)

    ag0 = ag(0)
    attn_half(0, top=True)
    wait(ag0)
    ag1 = ag(1)
    attn_half(1, top=True)
    attn_half(1, top=False)
    rs0 = rs(0)
    wait(ag1)
    ag2 = ag(2)
    attn_half(2, top=True)
    attn_half(2, top=False)
    wait(rs0)
    pacc_t[1] = rbuf_t[0] + pacc_t[2]
    pacc_b[1] = rbuf_b[0] + pacc_b[2]
    rs1 = rs(1)
    wait(ag2)
    attn_half(3, top=True)
    attn_half(3, top=False)
    wait(rs1)
    pacc_t[1] = rbuf_t[1] + pacc_t[3]
    pacc_b[1] = rbuf_b[1] + pacc_b[3]
    rs2 = rs(2)
    attn_half(0, top=False)
    wait(rs2)
    out_ref[0:HALF, :] = (rbuf_t[2].astype(jnp.float32)
                          + pacc_t[0].astype(jnp.float32))
    out_ref[HALF:, :] = (rbuf_b[2].astype(jnp.float32)
                         + pacc_b[0].astype(jnp.float32))


def kernel(x, Wq, K_ext, V_ext, Wo):
    i = lax.axis_index("i")
    x0 = x[0].astype(jnp.bfloat16)
    wq = (Wq * SCALE).astype(jnp.bfloat16)
    wo = Wo.astype(jnp.bfloat16)
    kt = lax.dynamic_slice_in_dim(K_ext, i * HQ, HQ, axis=2) \
        .astype(jnp.bfloat16).reshape(N_DEV, SQ, HQ * DH)
    vt = lax.dynamic_slice_in_dim(V_ext, i * HQ, HQ, axis=2) \
        .astype(jnp.bfloat16).reshape(N_DEV, SQ, HQ * DH)

    half = (HALF, DM)
    out2d = pl.pallas_call(
        _body,
        out_shape=jax.ShapeDtypeStruct((SQ, DM), jnp.float32),
        in_specs=[pl.BlockSpec(memory_space=pltpu.VMEM)] * 5,
        out_specs=pl.BlockSpec(memory_space=pltpu.VMEM),
        scratch_shapes=[
            pltpu.VMEM((N_DEV, *half), jnp.bfloat16),
            pltpu.VMEM((N_DEV, *half), jnp.bfloat16),
            pltpu.VMEM((N_DEV, *half), jnp.bfloat16),
            pltpu.VMEM((N_DEV, *half), jnp.bfloat16),
            pltpu.VMEM((N_DEV - 1, *half), jnp.bfloat16),
            pltpu.VMEM((N_DEV - 1, *half), jnp.bfloat16),
            pltpu.SemaphoreType.DMA((N_DEV - 1,)),
            pltpu.SemaphoreType.DMA((N_DEV - 1,)),
            pltpu.SemaphoreType.DMA((N_DEV - 1,)),
            pltpu.SemaphoreType.DMA((N_DEV - 1,)),
            pltpu.SemaphoreType.DMA((N_DEV - 1,)),
            pltpu.SemaphoreType.DMA((N_DEV - 1,)),
            pltpu.SemaphoreType.DMA((N_DEV - 1,)),
            pltpu.SemaphoreType.DMA((N_DEV - 1,)),
        ],
        compiler_params=pltpu.CompilerParams(
            collective_id=0, vmem_limit_bytes=42 * 1024 * 1024 + 512 * 1024),
    )(x0, wq, kt, vt, wo)
    return out2d[None].astype(jnp.float32)
